# Initial kernel scaffold; baseline (speedup 1.0000x reference)
import jax
import jax.numpy as jnp
from jax.experimental import pallas as pl


def kernel(x, edge_index, W_in, b_in, gammas, bn_gamma, bn_beta, W_up, b_up, alpha):
    raise NotImplementedError("write your pallas kernel here")


# speedup vs baseline: 13.9316x; 13.9316x over previous
import functools

import jax
import jax.numpy as jnp
from jax import lax
from jax.experimental import pallas as pl
from jax.experimental.pallas import tpu as pltpu
from jax.experimental.pallas import tpu_sc as plsc

NC = 2
NS = 16
TILES = NC * NS
W = 128
D = 128
N = 10000
NPAD = 10240
RPT = NPAD // NS
BR = 1000
GRID = N // BR

_mesh = plsc.VectorSubcoreMesh(core_axis_name="c", subcore_axis_name="s")


def _deg_body(dstw_hbm, deg_hbm, dst_v, ones_v, zb_v, acc1):
    c = lax.axis_index("c")
    s = lax.axis_index("s")
    w = c * NS + s
    nwin = dst_v.shape[0]
    for q in range(8):
        ones_v[pl.ds(q * 16, 16)] = jnp.ones((16,), jnp.float32)
        zb_v[pl.ds(q * 16, 16)] = jnp.zeros((16,), jnp.float32)
    for i in range(RPT // 128):
        pltpu.sync_copy(zb_v, acc1.at[pl.ds(s * RPT + i * 128, 128)])
    pltpu.sync_copy(dstw_hbm.at[w], dst_v)
    plsc.subcore_barrier()

    def body(j, carry):
        pltpu.sync_copy(ones_v, acc1.at[dst_v.at[j]], add=True)
        return carry

    lax.fori_loop(0, nwin, body, 0)
    plsc.subcore_barrier()
    pltpu.sync_copy(acc1.at[pl.ds(s * RPT, RPT)],
                    deg_hbm.at[c].at[pl.ds(s * RPT, RPT)])


def _segsum_body(t_hbm, srcw_hbm, dstw_hbm, m_hbm, src_v, dst_v, rows_v, zb_v,
                 acc, sem):
    c = lax.axis_index("c")
    s = lax.axis_index("s")
    w = c * NS + s
    nwin = src_v.shape[0]
    def zrow(i, carry):
        for q in range(8):
            zb_v[i, pl.ds(q * 16, 16)] = jnp.zeros((16,), jnp.float32)
        return carry
    lax.fori_loop(0, 64, zrow, 0)
    for i in range(RPT // 64):
        pltpu.sync_copy(zb_v, acc.at[pl.ds(s * RPT + i * 64, 64)])
    pltpu.sync_copy(srcw_hbm.at[w], src_v)
    pltpu.sync_copy(dstw_hbm.at[w], dst_v)
    plsc.subcore_barrier()

    def body(j, carry):
        pltpu.async_copy(t_hbm.at[src_v.at[j]], rows_v, sem).wait()
        pltpu.sync_copy(rows_v, acc.at[dst_v.at[j]], add=True)
        return carry

    lax.fori_loop(0, nwin, body, 0)
    plsc.subcore_barrier()
    pltpu.sync_copy(acc.at[pl.ds(s * RPT, RPT)],
                    m_hbm.at[c].at[pl.ds(s * RPT, RPT)])


def _make_deg(nwin):
    return pl.kernel(
        _deg_body,
        out_type=jax.ShapeDtypeStruct((NC, NPAD), jnp.float32),
        mesh=_mesh,
        scratch_types=[
            pltpu.VMEM((nwin, W), jnp.int32),
            pltpu.VMEM((W,), jnp.float32),
            pltpu.VMEM((W,), jnp.float32),
            pltpu.VMEM_SHARED((NPAD,), jnp.float32),
        ],
    )


def _make_segsum(nwin):
    return pl.kernel(
        _segsum_body,
        out_type=jax.ShapeDtypeStruct((NC, NPAD, D), jnp.float32),
        mesh=_mesh,
        scratch_types=[
            pltpu.VMEM((nwin, W), jnp.int32),
            pltpu.VMEM((nwin, W), jnp.int32),
            pltpu.VMEM((W, D), jnp.float32),
            pltpu.VMEM((64, D), jnp.float32),
            pltpu.VMEM_SHARED((NPAD, D), jnp.float32),
            pltpu.SemaphoreType.DMA,
        ],
    )


def _prep_body(x_r, w_r, b_r, d0_r, d1_r, g_r, h_o, t_o, out_o, r_o, a_o):
    deg = d0_r[...] + d1_r[...] + 1.0
    rv = lax.rsqrt(deg)
    av = 1.0 - 1.0 / deg
    h = jnp.dot(x_r[...], w_r[...], preferred_element_type=jnp.float32) + b_r[...]
    h_o[...] = h
    t_o[...] = rv * h
    out_o[...] = g_r[0, 0] * h
    r_o[...] = rv
    a_o[...] = av


def _iter_body(m0_r, m1_r, cur_r, out_r, r_r, a_r, g_r, cur_o, t_o, out_o):
    mm = m0_r[0] + m1_r[0]
    cnew = a_r[...] * cur_r[...] - r_r[...] * mm
    cur_o[...] = cnew
    t_o[...] = r_r[...] * cnew
    out_o[...] = out_r[...] + g_r[0, 0] * cnew


def _stats_body(o_r, ssum_o, ssq_o):
    @pl.when(pl.program_id(0) == 0)
    def _():
        ssum_o[...] = jnp.zeros_like(ssum_o)
        ssq_o[...] = jnp.zeros_like(ssq_o)

    blk = o_r[...]
    ssum_o[...] += jnp.sum(blk, axis=0, keepdims=True)
    ssq_o[...] += jnp.sum(blk * blk, axis=0, keepdims=True)


def _final_body(o_r, ssum_r, ssq_r, bg_r, bb_r, wu_r, bu_r, al_r, y_o):
    inv_n = 1.0 / N
    mean = ssum_r[...] * inv_n
    var = ssq_r[...] * inv_n - mean * mean
    xh = (o_r[...] - mean) * lax.rsqrt(var + 1e-5) * bg_r[...] + bb_r[...]
    y = jnp.dot(xh, wu_r[...], preferred_element_type=jnp.float32) + bu_r[...]
    y_o[...] = jnp.where(y >= 0, y, al_r[0, 0] * y)


def _rowspec(last):
    return pl.BlockSpec((BR, last), lambda i: (i, 0))


def _fullspec(shape):
    return pl.BlockSpec(shape, lambda i: tuple(0 for _ in shape))


def _sds(shape):
    return jax.ShapeDtypeStruct(shape, jnp.float32)


def kernel(x, edge_index, W_in, b_in, gammas, bn_gamma, bn_beta, W_up, b_up, alpha):
    n, d = x.shape
    e = edge_index.shape[1]
    k_order = gammas.shape[0] - 1

    src = edge_index[0].astype(jnp.int32)
    dst = edge_index[1].astype(jnp.int32)
    epad = -(-e // (TILES * W)) * (TILES * W)
    nwin = epad // (TILES * W)
    pad = epad - e
    pad_ar = jnp.arange(pad, dtype=jnp.int32)
    src_p = jnp.concatenate([src, pad_ar % 128])
    dst_p = jnp.concatenate([dst, n + pad_ar % (NPAD - n)])
    srcw = src_p.reshape(TILES, nwin, W)
    dstw = dst_p.reshape(TILES, nwin, W)

    degp = _make_deg(nwin)(dstw)
    d0 = degp[0, :n].reshape(n, 1)
    d1 = degp[1, :n].reshape(n, 1)

    prep = pl.pallas_call(
        _prep_body,
        grid=(GRID,),
        in_specs=[
            _rowspec(d), _fullspec((d, d)), _fullspec((1, d)),
            _rowspec(1), _rowspec(1), _fullspec((1, 1)),
        ],
        out_specs=[_rowspec(d), _rowspec(d), _rowspec(d), _rowspec(1), _rowspec(1)],
        out_shape=[_sds((n, d)), _sds((n, d)), _sds((n, d)), _sds((n, 1)), _sds((n, 1))],
    )
    cur, t, out, r, a = prep(x, W_in, b_in.reshape(1, d), d0, d1,
                             gammas[0].reshape(1, 1))

    segsum = _make_segsum(nwin)
    mspec = pl.BlockSpec((1, BR, D), lambda i: (0, i, 0))
    mspec1 = pl.BlockSpec((1, BR, D), lambda i: (1, i, 0))
    step = pl.pallas_call(
        _iter_body,
        grid=(GRID,),
        in_specs=[
            mspec, mspec1, _rowspec(d), _rowspec(d),
            _rowspec(1), _rowspec(1), _fullspec((1, 1)),
        ],
        out_specs=[_rowspec(d), _rowspec(d), _rowspec(d)],
        out_shape=[_sds((n, d)), _sds((n, d)), _sds((n, d))],
    )
    for k in range(1, k_order + 1):
        m = segsum(t, srcw, dstw)
        cur, t, out = step(m, m, cur, out, r, a, gammas[k].reshape(1, 1))

    stats = pl.pallas_call(
        _stats_body,
        grid=(GRID,),
        in_specs=[_rowspec(d)],
        out_specs=[_fullspec((1, d)), _fullspec((1, d))],
        out_shape=[_sds((1, d)), _sds((1, d))],
    )
    ssum, ssq = stats(out)

    final = pl.pallas_call(
        _final_body,
        grid=(GRID,),
        in_specs=[
            _rowspec(d), _fullspec((1, d)), _fullspec((1, d)),
            _fullspec((1, d)), _fullspec((1, d)), _fullspec((d, d)),
            _fullspec((1, d)), _fullspec((1, 1)),
        ],
        out_specs=[_rowspec(d)],
        out_shape=[_sds((n, d))],
    )
    (y,) = final(out, ssum, ssq, bn_gamma.reshape(1, d), bn_beta.reshape(1, d),
                 W_up, b_up.reshape(1, d), alpha.reshape(1, 1))
    return y


# speedup vs baseline: 18.4492x; 1.3243x over previous
import functools

import jax
import jax.numpy as jnp
from jax import lax
from jax.experimental import pallas as pl
from jax.experimental.pallas import tpu as pltpu
from jax.experimental.pallas import tpu_sc as plsc

NC = 2
NS = 16
TILES = NC * NS
W = 64
D = 128
N = 10000
NPAD = 10240
RPT = NPAD // NS
BR = 1000
GRID = N // BR

_mesh = plsc.VectorSubcoreMesh(core_axis_name="c", subcore_axis_name="s")


def _unpack_src(slab_v, j, wsrc_v):
    for q in range(W // 16):
        pk = slab_v[j, pl.ds(q * 16, 16)]
        wsrc_v[pl.ds(q * 16, 16)] = pk & 0xFFFF


def _unpack_dst(slab_v, j, wdst_v):
    for q in range(W // 16):
        pk = slab_v[j, pl.ds(q * 16, 16)]
        wdst_v[pl.ds(q * 16, 16)] = lax.shift_right_logical(pk, 16)


def _deg_body(pairw_hbm, deg_hbm, slab_v, ones_v, zb_v, wdst_v, acc1):
    c = lax.axis_index("c")
    s = lax.axis_index("s")
    w = c * NS + s
    nwin = slab_v.shape[0]
    for q in range(W // 16):
        ones_v[pl.ds(q * 16, 16)] = jnp.ones((16,), jnp.float32)
    for q in range(8):
        zb_v[pl.ds(q * 16, 16)] = jnp.zeros((16,), jnp.float32)
    for i in range(RPT // 128):
        pltpu.sync_copy(zb_v, acc1.at[pl.ds(s * RPT + i * 128, 128)])
    pltpu.sync_copy(pairw_hbm.at[w], slab_v)
    plsc.subcore_barrier()

    def body(j, carry):
        _unpack_dst(slab_v, j, wdst_v)
        pltpu.sync_copy(ones_v, acc1.at[wdst_v], add=True)
        return carry

    lax.fori_loop(0, nwin, body, 0)
    plsc.subcore_barrier()
    pltpu.sync_copy(acc1.at[pl.ds(s * RPT, RPT)],
                    deg_hbm.at[c].at[pl.ds(s * RPT, RPT)])


NBUF = 3


def _segsum_body(t_hbm, pairw_hbm, m_hbm, slab_v, rows0, rows1, rows2,
                 ws0, ws1, ws2, wd0, wd1, wd2, acc,
                 sg0, sg1, sg2, ss0, ss1, ss2):
    c = lax.axis_index("c")
    s = lax.axis_index("s")
    w = c * NS + s
    nwin = slab_v.shape[0]
    nsteps = nwin // NBUF
    bufs = [rows0, rows1, rows2]
    wsrcs = [ws0, ws1, ws2]
    wdsts = [wd0, wd1, wd2]
    sgs = [sg0, sg1, sg2]
    sss = [ss0, ss1, ss2]

    def zrow(i, carry):
        for q in range(8):
            rows0[i, pl.ds(q * 16, 16)] = jnp.zeros((16,), jnp.float32)
        return carry
    lax.fori_loop(0, W, zrow, 0)
    for i in range(RPT // W):
        pltpu.sync_copy(rows0, acc.at[pl.ds(s * RPT + i * W, W)])
    pltpu.sync_copy(pairw_hbm.at[w], slab_v)
    plsc.subcore_barrier()

    def start_gather(j, b):
        _unpack_src(slab_v, j, wsrcs[b])
        _unpack_dst(slab_v, j, wdsts[b])
        pltpu.async_copy(t_hbm.at[wsrcs[b]], bufs[b], sgs[b])

    def wait_gather(b):
        pltpu.make_async_copy(t_hbm.at[wsrcs[b]], bufs[b], sgs[b]).wait()

    def start_scat(b):
        pltpu.async_copy(bufs[b], acc.at[wdsts[b]], sss[b], add=True)

    def wait_scat(b):
        pltpu.make_async_copy(bufs[b], acc.at[wdsts[b]], sss[b]).wait()

    for b in range(NBUF):
        start_gather(b, b)

    def body(jj, carry):
        j = jj * NBUF
        for b in range(NBUF):
            wait_gather(b)
            start_scat(b)

        @pl.when(jj + 1 < nsteps)
        def _():
            for b in range(NBUF):
                wait_scat(b)
                start_gather(j + NBUF + b, b)
        return carry

    lax.fori_loop(0, nsteps, body, 0)
    for b in range(NBUF):
        wait_scat(b)
    plsc.subcore_barrier()
    pltpu.sync_copy(acc.at[pl.ds(s * RPT, RPT)],
                    m_hbm.at[c].at[pl.ds(s * RPT, RPT)])


def _make_deg(nwin):
    return pl.kernel(
        _deg_body,
        out_type=jax.ShapeDtypeStruct((NC, NPAD), jnp.float32),
        mesh=_mesh,
        scratch_types=[
            pltpu.VMEM((nwin, W), jnp.int32),
            pltpu.VMEM((W,), jnp.float32),
            pltpu.VMEM((128,), jnp.float32),
            pltpu.VMEM((W,), jnp.int32),
            pltpu.VMEM_SHARED((NPAD,), jnp.float32),
        ],
    )


def _make_segsum(nwin):
    return pl.kernel(
        _segsum_body,
        out_type=jax.ShapeDtypeStruct((NC, NPAD, D), jnp.float32),
        mesh=_mesh,
        scratch_types=(
            [pltpu.VMEM((nwin, W), jnp.int32)]
            + [pltpu.VMEM((W, D), jnp.float32)] * NBUF
            + [pltpu.VMEM((W,), jnp.int32)] * (2 * NBUF)
            + [pltpu.VMEM_SHARED((NPAD, D), jnp.float32)]
            + [pltpu.SemaphoreType.DMA] * (2 * NBUF)
        ),
    )


def _prep_body(x_r, w_r, b_r, d0_r, d1_r, g_r, h_o, t_o, out_o, r_o, a_o):
    deg = d0_r[...] + d1_r[...] + 1.0
    rv = lax.rsqrt(deg)
    av = 1.0 - 1.0 / deg
    h = jnp.dot(x_r[...], w_r[...], preferred_element_type=jnp.float32) + b_r[...]
    h_o[...] = h
    t_o[...] = rv * h
    out_o[...] = g_r[0, 0] * h
    r_o[...] = rv
    a_o[...] = av


def _iter_body(m0_r, m1_r, cur_r, out_r, r_r, a_r, g_r, cur_o, t_o, out_o):
    mm = m0_r[0] + m1_r[0]
    cnew = a_r[...] * cur_r[...] - r_r[...] * mm
    cur_o[...] = cnew
    t_o[...] = r_r[...] * cnew
    out_o[...] = out_r[...] + g_r[0, 0] * cnew


def _stats_body(o_r, ssum_o, ssq_o):
    @pl.when(pl.program_id(0) == 0)
    def _():
        ssum_o[...] = jnp.zeros_like(ssum_o)
        ssq_o[...] = jnp.zeros_like(ssq_o)

    blk = o_r[...]
    ssum_o[...] += jnp.sum(blk, axis=0, keepdims=True)
    ssq_o[...] += jnp.sum(blk * blk, axis=0, keepdims=True)


def _final_body(o_r, ssum_r, ssq_r, bg_r, bb_r, wu_r, bu_r, al_r, y_o):
    inv_n = 1.0 / N
    mean = ssum_r[...] * inv_n
    var = ssq_r[...] * inv_n - mean * mean
    xh = (o_r[...] - mean) * lax.rsqrt(var + 1e-5) * bg_r[...] + bb_r[...]
    y = jnp.dot(xh, wu_r[...], preferred_element_type=jnp.float32) + bu_r[...]
    y_o[...] = jnp.where(y >= 0, y, al_r[0, 0] * y)


def _rowspec(last):
    return pl.BlockSpec((BR, last), lambda i: (i, 0))


def _fullspec(shape):
    return pl.BlockSpec(shape, lambda i: tuple(0 for _ in shape))


def _sds(shape):
    return jax.ShapeDtypeStruct(shape, jnp.float32)


def kernel(x, edge_index, W_in, b_in, gammas, bn_gamma, bn_beta, W_up, b_up, alpha):
    n, d = x.shape
    e = edge_index.shape[1]
    k_order = gammas.shape[0] - 1

    src = edge_index[0].astype(jnp.int32)
    dst = edge_index[1].astype(jnp.int32)
    chunk = TILES * W * NBUF
    epad = -(-e // chunk) * chunk
    nwin = epad // (TILES * W)
    pad = epad - e
    pad_ar = jnp.arange(pad, dtype=jnp.int32)
    src_p = jnp.concatenate([src, pad_ar % 128])
    dst_p = jnp.concatenate([dst, n + pad_ar % (NPAD - n)])
    pairw = ((dst_p << 16) | src_p).reshape(TILES, nwin, W)

    degp = _make_deg(nwin)(pairw)
    d0 = degp[0, :n].reshape(n, 1)
    d1 = degp[1, :n].reshape(n, 1)

    prep = pl.pallas_call(
        _prep_body,
        grid=(GRID,),
        in_specs=[
            _rowspec(d), _fullspec((d, d)), _fullspec((1, d)),
            _rowspec(1), _rowspec(1), _fullspec((1, 1)),
        ],
        out_specs=[_rowspec(d), _rowspec(d), _rowspec(d), _rowspec(1), _rowspec(1)],
        out_shape=[_sds((n, d)), _sds((n, d)), _sds((n, d)), _sds((n, 1)), _sds((n, 1))],
    )
    cur, t, out, r, a = prep(x, W_in, b_in.reshape(1, d), d0, d1,
                             gammas[0].reshape(1, 1))

    segsum = _make_segsum(nwin)
    mspec = pl.BlockSpec((1, BR, D), lambda i: (0, i, 0))
    mspec1 = pl.BlockSpec((1, BR, D), lambda i: (1, i, 0))
    step = pl.pallas_call(
        _iter_body,
        grid=(GRID,),
        in_specs=[
            mspec, mspec1, _rowspec(d), _rowspec(d),
            _rowspec(1), _rowspec(1), _fullspec((1, 1)),
        ],
        out_specs=[_rowspec(d), _rowspec(d), _rowspec(d)],
        out_shape=[_sds((n, d)), _sds((n, d)), _sds((n, d))],
    )
    for k in range(1, k_order + 1):
        m = segsum(t, pairw)
        cur, t, out = step(m, m, cur, out, r, a, gammas[k].reshape(1, 1))

    stats = pl.pallas_call(
        _stats_body,
        grid=(GRID,),
        in_specs=[_rowspec(d)],
        out_specs=[_fullspec((1, d)), _fullspec((1, d))],
        out_shape=[_sds((1, d)), _sds((1, d))],
    )
    ssum, ssq = stats(out)

    final = pl.pallas_call(
        _final_body,
        grid=(GRID,),
        in_specs=[
            _rowspec(d), _fullspec((1, d)), _fullspec((1, d)),
            _fullspec((1, d)), _fullspec((1, d)), _fullspec((d, d)),
            _fullspec((1, d)), _fullspec((1, 1)),
        ],
        out_specs=[_rowspec(d)],
        out_shape=[_sds((n, d))],
    )
    (y,) = final(out, ssum, ssq, bn_gamma.reshape(1, d), bn_beta.reshape(1, d),
                 W_up, b_up.reshape(1, d), alpha.reshape(1, 1))
    return y
